# Initial kernel scaffold; baseline (speedup 1.0000x reference)
import jax
import jax.numpy as jnp
from jax.experimental import pallas as pl


def kernel(X, vertex, edges, X0, W1_w, W1_b, W2_w, W2_b, W3_w, W3_b):
    raise NotImplementedError("write your pallas kernel here")


# speedup vs baseline: 7.9322x; 7.9322x over previous
import functools

import jax
import jax.numpy as jnp
from jax import lax
from jax.experimental import pallas as pl
from jax.experimental.pallas import tpu as pltpu
from jax.experimental.pallas import tpu_sc as plsc

N_NODES = 10000
N_HEDGES = 5000
N_INC = 320000
HID = 128
ALPHA = 0.5

NC = 2
NS = 16
NW = NC * NS
CHUNK = 128
ROWS_PER_W = 80
ROWS2D = NW * ROWS_PER_W
INC_PAD = ROWS2D * CHUNK - N_INC

E_PAD = 5120
V_PAD = 10112
TBL_PAD = 16384
E_STRIPE = E_PAD // NS
V_STRIPE = V_PAD // NS

_MESH = plsc.VectorSubcoreMesh(core_axis_name="c", subcore_axis_name="s")


def _fill(buf, val):
  vec = jnp.full((16,), val, jnp.float32)

  def row(r, carry):
    for k in range(HID // 16):
      buf[r, pl.ds(k * 16, 16)] = vec
    return carry

  lax.fori_loop(0, CHUNK, row, 0)


def _zero_stripe(zbuf, sp, base, rows):
  done = 0
  while done < rows:
    n = min(CHUNK, rows - done)
    pltpu.sync_copy(zbuf.at[pl.ds(0, n)], sp.at[pl.ds(base + done, n)])
    done += n


@functools.partial(
    pl.kernel,
    out_type=jax.ShapeDtypeStruct((NC, E_PAD, HID), jnp.float32),
    mesh=_MESH,
    scratch_types=(
        pltpu.VMEM((ROWS_PER_W, CHUNK), jnp.int32),
        pltpu.VMEM((ROWS_PER_W, CHUNK), jnp.int32),
        pltpu.VMEM((CHUNK, HID), jnp.float32),
        pltpu.SemaphoreType.DMA,
        pltpu.VMEM_SHARED((E_PAD, HID), jnp.float32),
    ),
)
def _sc_pass1(x_hbm, v2d, e2d,
              esum_o,
              vidx, eidx, xrow, sem, esum_sp):
  c = lax.axis_index("c")
  s = lax.axis_index("s")
  wid = c * NS + s
  _fill(xrow, 0.0)
  _zero_stripe(xrow, esum_sp, s * E_STRIPE, E_STRIPE)
  pltpu.sync_copy(v2d.at[pl.ds(wid * ROWS_PER_W, ROWS_PER_W)], vidx)
  pltpu.sync_copy(e2d.at[pl.ds(wid * ROWS_PER_W, ROWS_PER_W)], eidx)
  plsc.subcore_barrier()

  def body(j, carry):
    pltpu.async_copy(x_hbm.at[vidx.at[j]], xrow, sem).wait()
    pltpu.sync_copy(xrow, esum_sp.at[eidx.at[j]], add=True)
    return carry

  lax.fori_loop(0, ROWS_PER_W, body, 0)
  plsc.subcore_barrier()
  pltpu.sync_copy(esum_sp.at[pl.ds(s * E_STRIPE, E_STRIPE)],
                  esum_o.at[c, pl.ds(s * E_STRIPE, E_STRIPE)])


@functools.partial(
    pl.kernel,
    out_type=jax.ShapeDtypeStruct((NC, V_PAD, HID), jnp.float32),
    mesh=_MESH,
    scratch_types=(
        pltpu.VMEM((ROWS_PER_W, CHUNK), jnp.int32),
        pltpu.VMEM((ROWS_PER_W, CHUNK), jnp.int32),
        pltpu.VMEM((CHUNK, HID), jnp.float32),
        pltpu.SemaphoreType.DMA,
        pltpu.VMEM_SHARED((V_PAD, HID), jnp.float32),
    ),
)
def _sc_pass2(g_hbm, v2d, e2d,
              vsum_o,
              vidx, eidx, grow, sem, vsum_sp):
  c = lax.axis_index("c")
  s = lax.axis_index("s")
  wid = c * NS + s
  _fill(grow, 0.0)
  _zero_stripe(grow, vsum_sp, s * V_STRIPE, V_STRIPE)
  pltpu.sync_copy(v2d.at[pl.ds(wid * ROWS_PER_W, ROWS_PER_W)], vidx)
  pltpu.sync_copy(e2d.at[pl.ds(wid * ROWS_PER_W, ROWS_PER_W)], eidx)
  plsc.subcore_barrier()

  def body(j, carry):
    pltpu.async_copy(g_hbm.at[eidx.at[j]], grow, sem).wait()
    pltpu.sync_copy(grow, vsum_sp.at[vidx.at[j]], add=True)
    return carry

  lax.fori_loop(0, ROWS_PER_W, body, 0)
  plsc.subcore_barrier()
  pltpu.sync_copy(vsum_sp.at[pl.ds(s * V_STRIPE, V_STRIPE)],
                  vsum_o.at[c, pl.ds(s * V_STRIPE, V_STRIPE)])


def _make_count_pass(n_pad, stripe):
  @functools.partial(
      pl.kernel,
      out_type=jax.ShapeDtypeStruct((NC, n_pad, HID), jnp.float32),
      mesh=_MESH,
      scratch_types=(
          pltpu.VMEM((ROWS_PER_W, CHUNK), jnp.int32),
          pltpu.VMEM((CHUNK, HID), jnp.float32),
          pltpu.VMEM_SHARED((n_pad, HID), jnp.float32),
      ),
  )
  def _count(i2d, cnt_o, idx, ones_v, cnt_sp):
    c = lax.axis_index("c")
    s = lax.axis_index("s")
    wid = c * NS + s
    _fill(ones_v, 0.0)
    _zero_stripe(ones_v, cnt_sp, s * stripe, stripe)
    _fill(ones_v, 1.0)
    pltpu.sync_copy(i2d.at[pl.ds(wid * ROWS_PER_W, ROWS_PER_W)], idx)
    plsc.subcore_barrier()

    def body(j, carry):
      pltpu.sync_copy(ones_v, cnt_sp.at[idx.at[j]], add=True)
      return carry

    lax.fori_loop(0, ROWS_PER_W, body, 0)
    plsc.subcore_barrier()
    pltpu.sync_copy(cnt_sp.at[pl.ds(s * stripe, stripe)],
                    cnt_o.at[c, pl.ds(s * stripe, stripe)])

  return _count


_sc_vcnt = _make_count_pass(V_PAD, V_STRIPE)
_sc_ecnt = _make_count_pass(E_PAD, E_STRIPE)


def _combine_body(es_ref, ec_ref, g_ref):
  e = es_ref[0] + es_ref[1]
  cnt = (ec_ref[0] + ec_ref[1])[:, 0:1]
  g_ref[...] = e / jnp.maximum(cnt, 1.0)


def _final_body(x_ref, x0_ref, vs_ref, vc_ref,
                w1_ref, b1_ref, w2_ref, b2_ref, w3_ref, b3_ref, o_ref):
  f32 = jnp.float32
  cnt = vc_ref[0, :N_NODES, 0:1] + vc_ref[1, :N_NODES, 0:1]
  inv = 1.0 / jnp.maximum(cnt, 1.0)
  S = (vs_ref[0, :N_NODES, :] + vs_ref[1, :N_NODES, :]) * inv
  T = jnp.dot(S, w1_ref[...], preferred_element_type=f32) + b1_ref[...]
  w2a = w2_ref[:HID, :]
  w2b = w2_ref[HID:, :]
  acc = (jnp.dot(x_ref[...], w2a, preferred_element_type=f32)
         + jnp.dot(T, w2b, preferred_element_type=f32) + b2_ref[...])
  Xv = acc * (cnt > 0.0).astype(f32)
  Xn = (1.0 - ALPHA) * Xv + ALPHA * x0_ref[...]
  o_ref[...] = jnp.dot(Xn, w3_ref[...], preferred_element_type=f32) + b3_ref[...]


def kernel(X, vertex, edges, X0, W1_w, W1_b, W2_w, W2_b, W3_w, W3_b):
  ar = jnp.arange(INC_PAD, dtype=jnp.int32)
  vpad_g = ar % N_NODES
  vpad_s = N_NODES + (ar % (V_PAD - N_NODES))
  epad = N_HEDGES + (ar % (E_PAD - N_HEDGES))
  v32 = vertex.astype(jnp.int32)
  e32 = edges.astype(jnp.int32)
  v2d_g = jnp.concatenate([v32, vpad_g]).reshape(ROWS2D, CHUNK)
  v2d_s = jnp.concatenate([v32, vpad_s]).reshape(ROWS2D, CHUNK)
  e2d = jnp.concatenate([e32, epad]).reshape(ROWS2D, CHUNK)

  Xbig = jnp.zeros((TBL_PAD, HID), jnp.float32).at[:N_NODES].set(X)
  esum = _sc_pass1(Xbig, v2d_g, e2d)
  vcnt = _sc_vcnt(v2d_s)
  ecnt = _sc_ecnt(e2d)

  G = pl.pallas_call(
      _combine_body,
      out_shape=jax.ShapeDtypeStruct((E_PAD, HID), jnp.float32),
  )(esum, ecnt)

  Gbig = jnp.zeros((TBL_PAD, HID), jnp.float32).at[:E_PAD].set(G)
  vsum = _sc_pass2(Gbig, v2d_s, e2d)

  out = pl.pallas_call(
      _final_body,
      out_shape=jax.ShapeDtypeStruct((N_NODES, HID), jnp.float32),
  )(X, X0, vsum, vcnt, W1_w, W1_b, W2_w, W2_b, W3_w, W3_b)
  return out


# speedup vs baseline: 9.1604x; 1.1548x over previous
import functools

import jax
import jax.numpy as jnp
from jax import lax
from jax.experimental import pallas as pl
from jax.experimental.pallas import tpu as pltpu
from jax.experimental.pallas import tpu_sc as plsc

N_NODES = 10000
N_HEDGES = 5000
N_INC = 320000
HID = 128
ALPHA = 0.5

NC = 2
NS = 16
NW = NC * NS
CHUNK = 128
ROWS_PER_W = 80
ROWS2D = NW * ROWS_PER_W
INC_PAD = ROWS2D * CHUNK - N_INC

E_PAD = 5120
V_PAD = 10112
TBL_PAD = 16384
E_STRIPE = E_PAD // NS
V_STRIPE = V_PAD // NS

_MESH = plsc.VectorSubcoreMesh(core_axis_name="c", subcore_axis_name="s")


def _fill(buf, val):
  vec = jnp.full((16,), val, jnp.float32)

  def row(r, carry):
    for k in range(HID // 16):
      buf[r, pl.ds(k * 16, 16)] = vec
    return carry

  lax.fori_loop(0, CHUNK, row, 0)


def _zero_stripe(zbuf, sp, base, rows):
  done = 0
  while done < rows:
    n = min(CHUNK, rows - done)
    pltpu.sync_copy(zbuf.at[pl.ds(0, n)], sp.at[pl.ds(base + done, n)])
    done += n


@functools.partial(
    pl.kernel,
    out_type=jax.ShapeDtypeStruct((NC, E_PAD, HID), jnp.float32),
    mesh=_MESH,
    scratch_types=(
        pltpu.VMEM((ROWS_PER_W, CHUNK), jnp.int32),
        pltpu.VMEM((ROWS_PER_W, CHUNK), jnp.int32),
        pltpu.VMEM((2 * CHUNK, HID), jnp.float32),
        pltpu.SemaphoreType.DMA,
        pltpu.VMEM_SHARED((E_PAD, HID), jnp.float32),
    ),
)
def _sc_pass1(x_hbm, v2d, e2d,
              esum_o,
              vidx, eidx, xbuf, sem, esum_sp):
  c = lax.axis_index("c")
  s = lax.axis_index("s")
  wid = c * NS + s
  _fill(xbuf, 0.0)
  _zero_stripe(xbuf, esum_sp, s * E_STRIPE, E_STRIPE)
  pltpu.sync_copy(v2d.at[pl.ds(wid * ROWS_PER_W, ROWS_PER_W)], vidx)
  pltpu.sync_copy(e2d.at[pl.ds(wid * ROWS_PER_W, ROWS_PER_W)], eidx)
  plsc.subcore_barrier()

  pltpu.async_copy(x_hbm.at[vidx.at[0]], xbuf.at[pl.ds(0, CHUNK)], sem)

  def body(j, carry):
    ph = lax.rem(j, 2) * CHUNK
    nph = lax.rem(j + 1, 2) * CHUNK

    @pl.when(j + 1 < ROWS_PER_W)
    def _():
      pltpu.async_copy(x_hbm.at[vidx.at[j + 1]], xbuf.at[pl.ds(nph, CHUNK)],
                       sem)

    pltpu.make_async_copy(x_hbm.at[vidx.at[0]],
                          xbuf.at[pl.ds(0, CHUNK)], sem).wait()
    pltpu.sync_copy(xbuf.at[pl.ds(ph, CHUNK)], esum_sp.at[eidx.at[j]],
                    add=True)
    return carry

  lax.fori_loop(0, ROWS_PER_W, body, 0)
  plsc.subcore_barrier()
  pltpu.sync_copy(esum_sp.at[pl.ds(s * E_STRIPE, E_STRIPE)],
                  esum_o.at[c, pl.ds(s * E_STRIPE, E_STRIPE)])


@functools.partial(
    pl.kernel,
    out_type=jax.ShapeDtypeStruct((NC, V_PAD, HID), jnp.float32),
    mesh=_MESH,
    scratch_types=(
        pltpu.VMEM((ROWS_PER_W, CHUNK), jnp.int32),
        pltpu.VMEM((ROWS_PER_W, CHUNK), jnp.int32),
        pltpu.VMEM((CHUNK, HID), jnp.float32),
        pltpu.SemaphoreType.DMA,
        pltpu.VMEM_SHARED((V_PAD, HID), jnp.float32),
    ),
)
def _sc_pass2(g_hbm, v2d, e2d,
              vsum_o,
              vidx, eidx, grow, sem, vsum_sp):
  c = lax.axis_index("c")
  s = lax.axis_index("s")
  wid = c * NS + s
  _fill(grow, 0.0)
  _zero_stripe(grow, vsum_sp, s * V_STRIPE, V_STRIPE)
  pltpu.sync_copy(v2d.at[pl.ds(wid * ROWS_PER_W, ROWS_PER_W)], vidx)
  pltpu.sync_copy(e2d.at[pl.ds(wid * ROWS_PER_W, ROWS_PER_W)], eidx)
  plsc.subcore_barrier()

  def body(j, carry):
    pltpu.async_copy(g_hbm.at[eidx.at[j]], grow, sem).wait()
    pltpu.sync_copy(grow, vsum_sp.at[vidx.at[j]], add=True)
    return carry

  lax.fori_loop(0, ROWS_PER_W, body, 0)
  plsc.subcore_barrier()
  pltpu.sync_copy(vsum_sp.at[pl.ds(s * V_STRIPE, V_STRIPE)],
                  vsum_o.at[c, pl.ds(s * V_STRIPE, V_STRIPE)])


def _make_count_pass(n_pad, stripe):
  @functools.partial(
      pl.kernel,
      out_type=jax.ShapeDtypeStruct((NC, n_pad, HID), jnp.float32),
      mesh=_MESH,
      scratch_types=(
          pltpu.VMEM((ROWS_PER_W, CHUNK), jnp.int32),
          pltpu.VMEM((CHUNK, HID), jnp.float32),
          pltpu.VMEM_SHARED((n_pad, HID), jnp.float32),
      ),
  )
  def _count(i2d, cnt_o, idx, ones_v, cnt_sp):
    c = lax.axis_index("c")
    s = lax.axis_index("s")
    wid = c * NS + s
    _fill(ones_v, 0.0)
    _zero_stripe(ones_v, cnt_sp, s * stripe, stripe)
    _fill(ones_v, 1.0)
    pltpu.sync_copy(i2d.at[pl.ds(wid * ROWS_PER_W, ROWS_PER_W)], idx)
    plsc.subcore_barrier()

    def body(j, carry):
      pltpu.sync_copy(ones_v, cnt_sp.at[idx.at[j]], add=True)
      return carry

    lax.fori_loop(0, ROWS_PER_W, body, 0)
    plsc.subcore_barrier()
    pltpu.sync_copy(cnt_sp.at[pl.ds(s * stripe, stripe)],
                    cnt_o.at[c, pl.ds(s * stripe, stripe)])

  return _count


_sc_vcnt = _make_count_pass(V_PAD, V_STRIPE)
_sc_ecnt = _make_count_pass(E_PAD, E_STRIPE)


def _combine_body(es_ref, ec_ref, g_ref):
  e = es_ref[0] + es_ref[1]
  cnt = (ec_ref[0] + ec_ref[1])[:, 0:1]
  g_ref[...] = e / jnp.maximum(cnt, 1.0)


def _final_body(x_ref, x0_ref, vs_ref, vc_ref,
                w1_ref, b1_ref, w2_ref, b2_ref, w3_ref, b3_ref, o_ref):
  f32 = jnp.float32
  cnt = vc_ref[0, :N_NODES, 0:1] + vc_ref[1, :N_NODES, 0:1]
  inv = 1.0 / jnp.maximum(cnt, 1.0)
  S = (vs_ref[0, :N_NODES, :] + vs_ref[1, :N_NODES, :]) * inv
  T = jnp.dot(S, w1_ref[...], preferred_element_type=f32) + b1_ref[...]
  w2a = w2_ref[:HID, :]
  w2b = w2_ref[HID:, :]
  acc = (jnp.dot(x_ref[...], w2a, preferred_element_type=f32)
         + jnp.dot(T, w2b, preferred_element_type=f32) + b2_ref[...])
  Xv = acc * (cnt > 0.0).astype(f32)
  Xn = (1.0 - ALPHA) * Xv + ALPHA * x0_ref[...]
  o_ref[...] = jnp.dot(Xn, w3_ref[...], preferred_element_type=f32) + b3_ref[...]


def kernel(X, vertex, edges, X0, W1_w, W1_b, W2_w, W2_b, W3_w, W3_b):
  ar = jnp.arange(INC_PAD, dtype=jnp.int32)
  vpad_g = ar % N_NODES
  vpad_s = N_NODES + (ar % (V_PAD - N_NODES))
  epad = N_HEDGES + (ar % (E_PAD - N_HEDGES))
  v32 = vertex.astype(jnp.int32)
  e32 = edges.astype(jnp.int32)
  v2d_g = jnp.concatenate([v32, vpad_g]).reshape(ROWS2D, CHUNK)
  v2d_s = jnp.concatenate([v32, vpad_s]).reshape(ROWS2D, CHUNK)
  e2d = jnp.concatenate([e32, epad]).reshape(ROWS2D, CHUNK)

  Xbig = jnp.zeros((TBL_PAD, HID), jnp.float32).at[:N_NODES].set(X)
  esum = _sc_pass1(Xbig, v2d_g, e2d)
  vcnt = _sc_vcnt(v2d_s)
  ecnt = _sc_ecnt(e2d)

  G = pl.pallas_call(
      _combine_body,
      out_shape=jax.ShapeDtypeStruct((E_PAD, HID), jnp.float32),
  )(esum, ecnt)

  Gbig = jnp.zeros((TBL_PAD, HID), jnp.float32).at[:E_PAD].set(G)
  vsum = _sc_pass2(Gbig, v2d_s, e2d)

  out = pl.pallas_call(
      _final_body,
      out_shape=jax.ShapeDtypeStruct((N_NODES, HID), jnp.float32),
  )(X, X0, vsum, vcnt, W1_w, W1_b, W2_w, W2_b, W3_w, W3_b)
  return out


# speedup vs baseline: 9.2065x; 1.0050x over previous
import functools

import jax
import jax.numpy as jnp
from jax import lax
from jax.experimental import pallas as pl
from jax.experimental.pallas import tpu as pltpu
from jax.experimental.pallas import tpu_sc as plsc

N_NODES = 10000
N_HEDGES = 5000
N_INC = 320000
HID = 128
ALPHA = 0.5

NC = 2
NS = 16
NW = NC * NS
CHUNK = 128
ROWS_PER_W = 80
ROWS2D = NW * ROWS_PER_W
INC_PAD = ROWS2D * CHUNK - N_INC

E_PAD = 5120
V_PAD = 10112
TBL_PAD = 16384
E_STRIPE = E_PAD // NS
V_STRIPE = V_PAD // NS

_MESH = plsc.VectorSubcoreMesh(core_axis_name="c", subcore_axis_name="s")


def _fill(buf, val):
  vec = jnp.full((16,), val, jnp.float32)

  def row(r, carry):
    for k in range(HID // 16):
      buf[r, pl.ds(k * 16, 16)] = vec
    return carry

  lax.fori_loop(0, CHUNK, row, 0)


def _zero_stripe(zbuf, sp, base, rows):
  done = 0
  while done < rows:
    n = min(CHUNK, rows - done)
    pltpu.sync_copy(zbuf.at[pl.ds(0, n)], sp.at[pl.ds(base + done, n)])
    done += n


@functools.partial(
    pl.kernel,
    out_type=jax.ShapeDtypeStruct((NC, E_PAD, HID), jnp.float32),
    mesh=_MESH,
    scratch_types=(
        pltpu.VMEM((ROWS_PER_W, CHUNK), jnp.int32),
        pltpu.VMEM((ROWS_PER_W, CHUNK), jnp.int32),
        pltpu.VMEM((2 * CHUNK, HID), jnp.float32),
        pltpu.SemaphoreType.DMA,
        pltpu.VMEM_SHARED((E_PAD, HID), jnp.float32),
    ),
)
def _sc_pass1(x_hbm, v2d, e2d,
              esum_o,
              vidx, eidx, xbuf, sem, esum_sp):
  c = lax.axis_index("c")
  s = lax.axis_index("s")
  wid = c * NS + s
  _fill(xbuf, 0.0)
  _zero_stripe(xbuf, esum_sp, s * E_STRIPE, E_STRIPE)
  pltpu.sync_copy(v2d.at[pl.ds(wid * ROWS_PER_W, ROWS_PER_W)], vidx)
  pltpu.sync_copy(e2d.at[pl.ds(wid * ROWS_PER_W, ROWS_PER_W)], eidx)
  plsc.subcore_barrier()

  pltpu.async_copy(x_hbm.at[vidx.at[0]], xbuf.at[pl.ds(0, CHUNK)], sem)

  def body(j, carry):
    ph = lax.rem(j, 2) * CHUNK
    nph = lax.rem(j + 1, 2) * CHUNK

    @pl.when(j + 1 < ROWS_PER_W)
    def _():
      pltpu.async_copy(x_hbm.at[vidx.at[j + 1]], xbuf.at[pl.ds(nph, CHUNK)],
                       sem)

    pltpu.make_async_copy(x_hbm.at[vidx.at[0]],
                          xbuf.at[pl.ds(0, CHUNK)], sem).wait()
    pltpu.sync_copy(xbuf.at[pl.ds(ph, CHUNK)], esum_sp.at[eidx.at[j]],
                    add=True)
    return carry

  lax.fori_loop(0, ROWS_PER_W, body, 0)
  plsc.subcore_barrier()
  pltpu.sync_copy(esum_sp.at[pl.ds(s * E_STRIPE, E_STRIPE)],
                  esum_o.at[c, pl.ds(s * E_STRIPE, E_STRIPE)])


@functools.partial(
    pl.kernel,
    out_type=jax.ShapeDtypeStruct((NC, V_PAD, HID), jnp.float32),
    mesh=_MESH,
    scratch_types=(
        pltpu.VMEM((ROWS_PER_W, CHUNK), jnp.int32),
        pltpu.VMEM((ROWS_PER_W, CHUNK), jnp.int32),
        pltpu.VMEM((CHUNK, HID), jnp.float32),
        pltpu.SemaphoreType.DMA,
        pltpu.VMEM_SHARED((V_PAD, HID), jnp.float32),
    ),
)
def _sc_pass2(g_hbm, v2d, e2d,
              vsum_o,
              vidx, eidx, grow, sem, vsum_sp):
  c = lax.axis_index("c")
  s = lax.axis_index("s")
  wid = c * NS + s
  _fill(grow, 0.0)
  _zero_stripe(grow, vsum_sp, s * V_STRIPE, V_STRIPE)
  pltpu.sync_copy(v2d.at[pl.ds(wid * ROWS_PER_W, ROWS_PER_W)], vidx)
  pltpu.sync_copy(e2d.at[pl.ds(wid * ROWS_PER_W, ROWS_PER_W)], eidx)
  plsc.subcore_barrier()

  def body(j, carry):
    pltpu.async_copy(g_hbm.at[eidx.at[j]], grow, sem).wait()
    pltpu.sync_copy(grow, vsum_sp.at[vidx.at[j]], add=True)
    return carry

  lax.fori_loop(0, ROWS_PER_W, body, 0)
  plsc.subcore_barrier()
  pltpu.sync_copy(vsum_sp.at[pl.ds(s * V_STRIPE, V_STRIPE)],
                  vsum_o.at[c, pl.ds(s * V_STRIPE, V_STRIPE)])


def _make_count_pass(n_pad, stripe):
  @functools.partial(
      pl.kernel,
      out_type=jax.ShapeDtypeStruct((NC, n_pad, HID), jnp.float32),
      mesh=_MESH,
      scratch_types=(
          pltpu.VMEM((ROWS_PER_W, CHUNK), jnp.int32),
          pltpu.VMEM((CHUNK, HID), jnp.float32),
          pltpu.SemaphoreType.DMA,
          pltpu.VMEM_SHARED((n_pad, HID), jnp.float32),
      ),
  )
  def _count(i2d, cnt_o, idx, ones_v, sem, cnt_sp):
    c = lax.axis_index("c")
    s = lax.axis_index("s")
    wid = c * NS + s
    _fill(ones_v, 0.0)
    _zero_stripe(ones_v, cnt_sp, s * stripe, stripe)
    _fill(ones_v, 1.0)
    pltpu.sync_copy(i2d.at[pl.ds(wid * ROWS_PER_W, ROWS_PER_W)], idx)
    plsc.subcore_barrier()

    pltpu.async_copy(ones_v, cnt_sp.at[idx.at[0]], sem, add=True)

    def body(j, carry):
      @pl.when(j + 1 < ROWS_PER_W)
      def _():
        pltpu.async_copy(ones_v, cnt_sp.at[idx.at[j + 1]], sem, add=True)

      pltpu.make_async_copy(ones_v, cnt_sp.at[idx.at[0]], sem).wait()
      return carry

    lax.fori_loop(0, ROWS_PER_W, body, 0)
    plsc.subcore_barrier()
    pltpu.sync_copy(cnt_sp.at[pl.ds(s * stripe, stripe)],
                    cnt_o.at[c, pl.ds(s * stripe, stripe)])

  return _count


_sc_vcnt = _make_count_pass(V_PAD, V_STRIPE)
_sc_ecnt = _make_count_pass(E_PAD, E_STRIPE)


def _combine_body(es_ref, ec_ref, g_ref):
  e = es_ref[0] + es_ref[1]
  cnt = (ec_ref[0] + ec_ref[1])[:, 0:1]
  g_ref[...] = e / jnp.maximum(cnt, 1.0)


def _final_body(x_ref, x0_ref, vs_ref, vc_ref,
                w1_ref, b1_ref, w2_ref, b2_ref, w3_ref, b3_ref, o_ref):
  f32 = jnp.float32
  cnt = vc_ref[0, :N_NODES, 0:1] + vc_ref[1, :N_NODES, 0:1]
  inv = 1.0 / jnp.maximum(cnt, 1.0)
  S = (vs_ref[0, :N_NODES, :] + vs_ref[1, :N_NODES, :]) * inv
  T = jnp.dot(S, w1_ref[...], preferred_element_type=f32) + b1_ref[...]
  w2a = w2_ref[:HID, :]
  w2b = w2_ref[HID:, :]
  acc = (jnp.dot(x_ref[...], w2a, preferred_element_type=f32)
         + jnp.dot(T, w2b, preferred_element_type=f32) + b2_ref[...])
  Xv = acc * (cnt > 0.0).astype(f32)
  Xn = (1.0 - ALPHA) * Xv + ALPHA * x0_ref[...]
  o_ref[...] = jnp.dot(Xn, w3_ref[...], preferred_element_type=f32) + b3_ref[...]


def kernel(X, vertex, edges, X0, W1_w, W1_b, W2_w, W2_b, W3_w, W3_b):
  ar = jnp.arange(INC_PAD, dtype=jnp.int32)
  vpad_g = ar % N_NODES
  vpad_s = N_NODES + (ar % (V_PAD - N_NODES))
  epad = N_HEDGES + (ar % (E_PAD - N_HEDGES))
  v32 = vertex.astype(jnp.int32)
  e32 = edges.astype(jnp.int32)
  v2d_g = jnp.concatenate([v32, vpad_g]).reshape(ROWS2D, CHUNK)
  v2d_s = jnp.concatenate([v32, vpad_s]).reshape(ROWS2D, CHUNK)
  e2d = jnp.concatenate([e32, epad]).reshape(ROWS2D, CHUNK)

  Xbig = jnp.zeros((TBL_PAD, HID), jnp.float32).at[:N_NODES].set(X)
  esum = _sc_pass1(Xbig, v2d_g, e2d)
  vcnt = _sc_vcnt(v2d_s)
  ecnt = _sc_ecnt(e2d)

  G = pl.pallas_call(
      _combine_body,
      out_shape=jax.ShapeDtypeStruct((E_PAD, HID), jnp.float32),
  )(esum, ecnt)

  Gbig = jnp.zeros((TBL_PAD, HID), jnp.float32).at[:E_PAD].set(G)
  vsum = _sc_pass2(Gbig, v2d_s, e2d)

  out = pl.pallas_call(
      _final_body,
      out_shape=jax.ShapeDtypeStruct((N_NODES, HID), jnp.float32),
  )(X, X0, vsum, vcnt, W1_w, W1_b, W2_w, W2_b, W3_w, W3_b)
  return out
